# baseline (speedup 1.0000x reference)
import functools

import jax
import jax.numpy as jnp
from jax import lax
from jax.experimental import pallas as pl
from jax.experimental.pallas import tpu as pltpu
from jax.experimental.pallas import tpu_sc as plsc

_F = 26
_B = 4096
_L = 20
_V = 100000
_D = 64

_NW = 32
_FB = _B * _F
_BAGS_W = _FB // _NW
_NB = 32
_ROWS_C = _NB * _L
_GROWS = 128
_NG = _ROWS_C // _GROWS
_NCHUNK = _BAGS_W // _NB
_NLANE = 16


@functools.partial(
    pl.kernel,
    mesh=plsc.VectorSubcoreMesh(core_axis_name="c", subcore_axis_name="s"),
    out_type=jax.ShapeDtypeStruct((_FB, _D), jnp.float32),
    compiler_params=pltpu.CompilerParams(use_tc_tiling_on_sc=False),
    scratch_types=[
        pltpu.VMEM((2, _NB, _L), jnp.int32),
        pltpu.VMEM((2, _ROWS_C, _D), jnp.float32),
        pltpu.VMEM((_NB, _D), jnp.float32),
        pltpu.VMEM((_NB,), jnp.int32),
        pltpu.SemaphoreType.DMA,
        pltpu.SemaphoreType.DMA,
    ],
)
def _sc_lookup(idx_hbm, tab_hbm, out_hbm, idxv, rowsv, outv, oidx, gsem0, gsem1):
    gsem = (gsem0, gsem1)
    wid = lax.axis_index("c") * 16 + lax.axis_index("s")
    g0 = wid * _BAGS_W

    def fire(cn, s):
        base_g = g0 + cn * _NB
        f_s = base_g >> 12
        pltpu.sync_copy(idx_hbm.at[pl.ds(base_g, _NB)], idxv.at[s])
        for j in range(_NB):
            pltpu.async_copy(
                tab_hbm.at[f_s].at[idxv.at[s, j]],
                rowsv.at[s, pl.ds(j * _L, _L)],
                gsem[s],
            )

    def drain(s):
        pltpu.make_async_copy(
            tab_hbm.at[0].at[pl.ds(0, _ROWS_C)], rowsv.at[s], gsem[s]
        ).wait()

    def consume(cn, s):
        def bag(j, carry):
            base = j * _L
            for k in range(_D // _NLANE):
                acc = rowsv[s, base, pl.ds(k * _NLANE, _NLANE)]
                for l in range(1, _L):
                    acc = acc + rowsv[s, base + l, pl.ds(k * _NLANE, _NLANE)]
                outv[j, pl.ds(k * _NLANE, _NLANE)] = acc
            return carry

        lax.fori_loop(0, _NB, bag, 0)

        base_g = g0 + cn * _NB
        f_s = base_g >> 12
        b0 = base_g & (_B - 1)
        i16 = lax.iota(jnp.int32, _NLANE)
        oidx[pl.ds(0, _NLANE)] = (b0 + i16) * _F + f_s
        oidx[pl.ds(_NLANE, _NLANE)] = (b0 + _NLANE + i16) * _F + f_s
        pltpu.sync_copy(outv, out_hbm.at[oidx])

    fire(0, 0)
    fire(1, 1)

    def step(c2, carry):
        for s in range(2):
            cn = 2 * c2 + s
            drain(s)
            consume(cn, s)

            @pl.when(cn + 2 < _NCHUNK)
            def _():
                fire(cn + 2, s)

        return carry

    lax.fori_loop(0, _NCHUNK // 2, step, 0)


def kernel(indices, tables):
    idx2 = indices.astype(jnp.int32).reshape(_F * _B, _L)
    out = _sc_lookup(idx2, tables)
    return out.reshape(_B, _F * _D)


# speedup vs baseline: 1.0339x; 1.0339x over previous
import functools

import jax
import jax.numpy as jnp
from jax import lax
from jax.experimental import pallas as pl
from jax.experimental.pallas import tpu as pltpu
from jax.experimental.pallas import tpu_sc as plsc

_F = 26
_B = 4096
_L = 20
_V = 100000
_D = 64

_NW = 32
_FB = _B * _F
_BAGS_W = _FB // _NW
_NB = 32
_ROWS_C = _NB * _L
_NCHUNK = _BAGS_W // _NB
_NLANE = 16
_NBUF = 3
_UNROLL = 6
_MAIN = _NCHUNK - _NCHUNK % _UNROLL


@functools.partial(
    pl.kernel,
    mesh=plsc.VectorSubcoreMesh(core_axis_name="c", subcore_axis_name="s"),
    out_type=jax.ShapeDtypeStruct((_FB, _D), jnp.float32),
    compiler_params=pltpu.CompilerParams(use_tc_tiling_on_sc=False),
    scratch_types=[
        pltpu.VMEM((_NBUF, _NB, _L), jnp.int32),
        pltpu.VMEM((_NBUF, _ROWS_C, _D), jnp.float32),
        pltpu.VMEM((2, _NB, _D), jnp.float32),
        pltpu.VMEM((2, _NB), jnp.int32),
        pltpu.SemaphoreType.DMA,
        pltpu.SemaphoreType.DMA,
        pltpu.SemaphoreType.DMA,
        pltpu.SemaphoreType.DMA,
        pltpu.SemaphoreType.DMA,
        pltpu.SemaphoreType.DMA,
        pltpu.SemaphoreType.DMA,
        pltpu.SemaphoreType.DMA,
    ],
)
def _sc_lookup(idx_hbm, tab_hbm, out_hbm, idxv, rowsv, outv, oidx,
               g0sem, g1sem, g2sem, i0sem, i1sem, i2sem, o0sem, o1sem):
    gsem = (g0sem, g1sem, g2sem)
    isem = (i0sem, i1sem, i2sem)
    osem = (o0sem, o1sem)
    wid = lax.axis_index("c") * 16 + lax.axis_index("s")
    g0 = wid * _BAGS_W

    def fire_idx(cn, b):
        pltpu.async_copy(idx_hbm.at[pl.ds(g0 + cn * _NB, _NB)], idxv.at[b],
                         isem[b])

    def wait_idx(b):
        pltpu.make_async_copy(idx_hbm.at[pl.ds(0, _NB)], idxv.at[b],
                              isem[b]).wait()

    def fire_gathers(cn, b):
        f_s = (g0 + cn * _NB) >> 12
        for j in range(_NB):
            pltpu.async_copy(
                tab_hbm.at[f_s].at[idxv.at[b, j]],
                rowsv.at[b, pl.ds(j * _L, _L)],
                gsem[b],
            )

    def wait_gathers(b):
        pltpu.make_async_copy(
            tab_hbm.at[0].at[pl.ds(0, _ROWS_C)], rowsv.at[b], gsem[b]
        ).wait()

    def wait_out(cn, os):
        pltpu.make_async_copy(outv.at[os], out_hbm.at[oidx.at[os]],
                              osem[os]).wait()

    def consume(cn, b, os):
        def bag(j, carry):
            base = j * _L
            for k in range(_D // _NLANE):
                acc = rowsv[b, base, pl.ds(k * _NLANE, _NLANE)]
                for l in range(1, _L):
                    acc = acc + rowsv[b, base + l, pl.ds(k * _NLANE, _NLANE)]
                outv[os, j, pl.ds(k * _NLANE, _NLANE)] = acc
            return carry

        lax.fori_loop(0, _NB, bag, 0)

        base_g = g0 + cn * _NB
        f_s = base_g >> 12
        b0 = base_g & (_B - 1)
        i16 = lax.iota(jnp.int32, _NLANE)
        oidx[os, pl.ds(0, _NLANE)] = (b0 + i16) * _F + f_s
        oidx[os, pl.ds(_NLANE, _NLANE)] = (b0 + _NLANE + i16) * _F + f_s
        pltpu.async_copy(outv.at[os], out_hbm.at[oidx.at[os]], osem[os])

    def pipeline_step(cn, b, os):
        wait_gathers(b)

        @pl.when(cn + _NBUF < _NCHUNK)
        def _():
            fire_idx(cn + _NBUF, b)

        @pl.when(cn >= 2)
        def _():
            wait_out(cn - 2, os)

        consume(cn, b, os)

        @pl.when(cn + _NBUF < _NCHUNK)
        def _():
            wait_idx(b)
            fire_gathers(cn + _NBUF, b)

    for b in range(_NBUF):
        fire_idx(b, b)
    for b in range(_NBUF):
        wait_idx(b)
        fire_gathers(b, b)

    def step(c6, carry):
        for u in range(_UNROLL):
            cn = _UNROLL * c6 + u
            pipeline_step(cn, u % _NBUF, u % 2)
        return carry

    lax.fori_loop(0, _MAIN // _UNROLL, step, 0)

    for cn in range(_MAIN, _NCHUNK):
        wait_gathers(cn % _NBUF)
        wait_out(cn - 2, cn % 2)
        consume(cn, cn % _NBUF, cn % 2)

    wait_out(_NCHUNK - 2, (_NCHUNK - 2) % 2)
    wait_out(_NCHUNK - 1, (_NCHUNK - 1) % 2)


def kernel(indices, tables):
    idx2 = indices.astype(jnp.int32).reshape(_F * _B, _L)
    out = _sc_lookup(idx2, tables)
    return out.reshape(_B, _F * _D)


# speedup vs baseline: 1.0361x; 1.0021x over previous
import functools

import jax
import jax.numpy as jnp
from jax import lax
from jax.experimental import pallas as pl
from jax.experimental.pallas import tpu as pltpu
from jax.experimental.pallas import tpu_sc as plsc

_F = 26
_B = 4096
_L = 20
_V = 100000
_D = 64

_NW = 32
_FB = _B * _F
_BAGS_W = _FB // _NW
_NB = 32
_ROWS_C = _NB * _L
_NCHUNK = _BAGS_W // _NB
_NLANE = 16
_NBUF = 3
_UNROLL = 6
_MAIN = _NCHUNK - _NCHUNK % _UNROLL


@functools.partial(
    pl.kernel,
    mesh=plsc.VectorSubcoreMesh(core_axis_name="c", subcore_axis_name="s"),
    out_type=jax.ShapeDtypeStruct((_FB, _D), jnp.float32),
    compiler_params=pltpu.CompilerParams(use_tc_tiling_on_sc=False),
    scratch_types=[
        pltpu.VMEM((_NBUF, _NB, _L), jnp.int32),
        pltpu.VMEM((_NBUF, _ROWS_C, _D), jnp.float32),
        pltpu.VMEM((2, _NB, _D), jnp.float32),
        pltpu.VMEM((2, _NB), jnp.int32),
        pltpu.SemaphoreType.DMA,
        pltpu.SemaphoreType.DMA,
        pltpu.SemaphoreType.DMA,
        pltpu.SemaphoreType.DMA,
        pltpu.SemaphoreType.DMA,
        pltpu.SemaphoreType.DMA,
        pltpu.SemaphoreType.DMA,
        pltpu.SemaphoreType.DMA,
    ],
)
def _sc_lookup(idx_hbm, tab_hbm, out_hbm, idxv, rowsv, outv, oidx,
               g0sem, g1sem, g2sem, i0sem, i1sem, i2sem, o0sem, o1sem):
    gsem = (g0sem, g1sem, g2sem)
    isem = (i0sem, i1sem, i2sem)
    osem = (o0sem, o1sem)
    wid = lax.axis_index("c") * 16 + lax.axis_index("s")
    g0 = wid * _BAGS_W

    def fire_idx(cn, b):
        base_g = g0 + cn * _NB
        f_s = base_g >> 12
        b0 = base_g & (_B - 1)
        pltpu.async_copy(idx_hbm.at[f_s].at[pl.ds(b0, _NB)], idxv.at[b],
                         isem[b])

    def wait_idx(b):
        pltpu.make_async_copy(idx_hbm.at[0].at[pl.ds(0, _NB)], idxv.at[b],
                              isem[b]).wait()

    def fire_gathers(cn, b):
        f_s = (g0 + cn * _NB) >> 12
        for j in range(_NB):
            pltpu.async_copy(
                tab_hbm.at[f_s].at[idxv.at[b, j]],
                rowsv.at[b, pl.ds(j * _L, _L)],
                gsem[b],
            )

    def wait_gathers(b):
        pltpu.make_async_copy(
            tab_hbm.at[0].at[pl.ds(0, _ROWS_C)], rowsv.at[b], gsem[b]
        ).wait()

    def wait_out(cn, os):
        pltpu.make_async_copy(outv.at[os], out_hbm.at[oidx.at[os]],
                              osem[os]).wait()

    def consume(cn, b, os):
        def bag(j, carry):
            base = j * _L
            for k in range(_D // _NLANE):
                acc = rowsv[b, base, pl.ds(k * _NLANE, _NLANE)]
                for l in range(1, _L):
                    acc = acc + rowsv[b, base + l, pl.ds(k * _NLANE, _NLANE)]
                outv[os, j, pl.ds(k * _NLANE, _NLANE)] = acc
            return carry

        lax.fori_loop(0, _NB, bag, 0)

        base_g = g0 + cn * _NB
        f_s = base_g >> 12
        b0 = base_g & (_B - 1)
        i16 = lax.iota(jnp.int32, _NLANE)
        oidx[os, pl.ds(0, _NLANE)] = (b0 + i16) * _F + f_s
        oidx[os, pl.ds(_NLANE, _NLANE)] = (b0 + _NLANE + i16) * _F + f_s
        pltpu.async_copy(outv.at[os], out_hbm.at[oidx.at[os]], osem[os])

    def pipeline_step(cn, b, os):
        wait_gathers(b)

        @pl.when(cn + _NBUF < _NCHUNK)
        def _():
            fire_idx(cn + _NBUF, b)

        @pl.when(cn >= 2)
        def _():
            wait_out(cn - 2, os)

        consume(cn, b, os)

        @pl.when(cn + _NBUF < _NCHUNK)
        def _():
            wait_idx(b)
            fire_gathers(cn + _NBUF, b)

    for b in range(_NBUF):
        fire_idx(b, b)
    for b in range(_NBUF):
        wait_idx(b)
        fire_gathers(b, b)

    def step(c6, carry):
        for u in range(_UNROLL):
            cn = _UNROLL * c6 + u
            pipeline_step(cn, u % _NBUF, u % 2)
        return carry

    lax.fori_loop(0, _MAIN // _UNROLL, step, 0)

    for cn in range(_MAIN, _NCHUNK):
        wait_gathers(cn % _NBUF)
        wait_out(cn - 2, cn % 2)
        consume(cn, cn % _NBUF, cn % 2)

    wait_out(_NCHUNK - 2, (_NCHUNK - 2) % 2)
    wait_out(_NCHUNK - 1, (_NCHUNK - 1) % 2)


def kernel(indices, tables):
    out = _sc_lookup(indices.astype(jnp.int32), tables)
    return out.reshape(_B, _F * _D)
